# Initial kernel scaffold; baseline (speedup 1.0000x reference)
import jax
import jax.numpy as jnp
from jax.experimental import pallas as pl


def kernel(xyz1, xyz2, points1, points2, W1, b1, W2, b2):
    raise NotImplementedError("write your pallas kernel here")


# speedup vs baseline: 24.9741x; 24.9741x over previous
import functools

import jax
import jax.numpy as jnp
from jax.experimental import pallas as pl

BN = 256
N2 = 1024
C2 = 256
C1 = 128
BIG_I = 1 << 30
INF = 3e38


def _fp_kernel(xyz1_ref, xyz2t_ref, p1_ref, p2_ref, w1a_ref, w1b_ref,
               w2_ref, b1_ref, b2_ref, out_ref):
    x1 = xyz1_ref[0]
    x2t = xyz2t_ref[0]
    sq1 = jnp.sum(x1 * x1, axis=1, keepdims=True)
    sq2 = jnp.sum(x2t * x2t, axis=0, keepdims=True)
    dot = jax.lax.dot_general(x1, x2t, (((1,), (0,)), ((), ())),
                              preferred_element_type=jnp.float32)
    d = sq1 + sq2 - 2.0 * dot

    cidx = jax.lax.broadcasted_iota(jnp.int32, (BN, N2), 1)

    def take_min(dm):
        m = jnp.min(dm, axis=1, keepdims=True)
        i = jnp.min(jnp.where(dm == m, cidx, BIG_I), axis=1, keepdims=True)
        return m, i, jnp.where(cidx == i, INF, dm)

    m1, i1, d = take_min(d)
    m2, i2, d = take_min(d)
    m3, i3, _ = take_min(d)

    r1 = 1.0 / jnp.maximum(m1, 1e-10)
    r2 = 1.0 / jnp.maximum(m2, 1e-10)
    r3 = 1.0 / jnp.maximum(m3, 1e-10)
    norm = r1 + r2 + r3
    w1 = r1 / norm
    w2 = r2 / norm
    w3 = r3 / norm

    zero = jnp.float32(0.0)
    m = (jnp.where(cidx == i1, w1, zero)
         + jnp.where(cidx == i2, w2, zero)
         + jnp.where(cidx == i3, w3, zero))
    interp = jax.lax.dot_general(m, p2_ref[0], (((1,), (0,)), ((), ())),
                                 preferred_element_type=jnp.float32)

    h = interp @ w1a_ref[...] + p1_ref[0] @ w1b_ref[...] + b1_ref[...]
    h = jnp.maximum(h, 0.0)
    o = h @ w2_ref[...] + b2_ref[...]
    out_ref[0] = jnp.maximum(o, 0.0)


@jax.jit
def kernel(xyz1, xyz2, points1, points2, W1, b1, W2, b2):
    B, N1, _ = xyz1.shape
    xyz1p = jnp.pad(xyz1, ((0, 0), (0, 0), (0, 5)))
    xyz2t = jnp.pad(xyz2, ((0, 0), (0, 0), (0, 5))).transpose(0, 2, 1)
    w1a = W1[:C2]
    w1b = W1[C2:]

    grid = (B, N1 // BN)
    out = pl.pallas_call(
        _fp_kernel,
        grid=grid,
        in_specs=[
            pl.BlockSpec((1, BN, 8), lambda b, n: (b, n, 0)),
            pl.BlockSpec((1, 8, N2), lambda b, n: (b, 0, 0)),
            pl.BlockSpec((1, BN, C1), lambda b, n: (b, n, 0)),
            pl.BlockSpec((1, N2, C2), lambda b, n: (b, 0, 0)),
            pl.BlockSpec((C2, C2), lambda b, n: (0, 0)),
            pl.BlockSpec((C1, C2), lambda b, n: (0, 0)),
            pl.BlockSpec((C2, C2), lambda b, n: (0, 0)),
            pl.BlockSpec((1, C2), lambda b, n: (0, 0)),
            pl.BlockSpec((1, C2), lambda b, n: (0, 0)),
        ],
        out_specs=pl.BlockSpec((1, BN, C2), lambda b, n: (b, n, 0)),
        out_shape=jax.ShapeDtypeStruct((B, N1, C2), jnp.float32),
    )(xyz1p, xyz2t, points1, points2, w1a, w1b, W2,
      b1.reshape(1, C2), b2.reshape(1, C2))
    return out


# speedup vs baseline: 35.5297x; 1.4227x over previous
import functools

import jax
import jax.numpy as jnp
from jax.experimental import pallas as pl

BN = 256
N2 = 1024
C2 = 256
C1 = 128
BIG_I = 1 << 30
INF = 3e38


def _fp_kernel(xyz1_ref, xyz2t_ref, p1_ref, p2_ref, w1a_ref, w1b_ref,
               w2_ref, b1_ref, b2_ref, out_ref):
    x1 = xyz1_ref[0]
    x2t = xyz2t_ref[0]
    sq1 = jnp.sum(x1 * x1, axis=1, keepdims=True)
    sq2 = jnp.sum(x2t * x2t, axis=0, keepdims=True)
    dot = jax.lax.dot_general(x1, x2t, (((1,), (0,)), ((), ())),
                              preferred_element_type=jnp.float32)
    d = sq1 + sq2 - 2.0 * dot

    m1 = jnp.min(d, axis=1, keepdims=True)
    e = jnp.where(d == m1, INF, d)
    m2 = jnp.min(e, axis=1, keepdims=True)
    f = jnp.where(e == m2, INF, e)
    m3 = jnp.min(f, axis=1, keepdims=True)

    r1 = 1.0 / jnp.maximum(m1, 1e-10)
    r2 = 1.0 / jnp.maximum(m2, 1e-10)
    r3 = 1.0 / jnp.maximum(m3, 1e-10)
    norm = r1 + r2 + r3
    w1 = r1 / norm
    w2 = r2 / norm
    w3 = r3 / norm

    zero = jnp.float32(0.0)
    m = (jnp.where(d == m1, w1, zero)
         + jnp.where(d == m2, w2, zero)
         + jnp.where(d == m3, w3, zero))
    interp = jax.lax.dot_general(m, p2_ref[0], (((1,), (0,)), ((), ())),
                                 preferred_element_type=jnp.float32)

    h = interp @ w1a_ref[...] + p1_ref[0] @ w1b_ref[...] + b1_ref[...]
    h = jnp.maximum(h, 0.0)
    o = h @ w2_ref[...] + b2_ref[...]
    out_ref[0] = jnp.maximum(o, 0.0)


@jax.jit
def kernel(xyz1, xyz2, points1, points2, W1, b1, W2, b2):
    B, N1, _ = xyz1.shape
    xyz1p = jnp.pad(xyz1, ((0, 0), (0, 0), (0, 5)))
    xyz2t = jnp.pad(xyz2, ((0, 0), (0, 0), (0, 5))).transpose(0, 2, 1)
    w1a = W1[:C2]
    w1b = W1[C2:]

    grid = (B, N1 // BN)
    out = pl.pallas_call(
        _fp_kernel,
        grid=grid,
        in_specs=[
            pl.BlockSpec((1, BN, 8), lambda b, n: (b, n, 0)),
            pl.BlockSpec((1, 8, N2), lambda b, n: (b, 0, 0)),
            pl.BlockSpec((1, BN, C1), lambda b, n: (b, n, 0)),
            pl.BlockSpec((1, N2, C2), lambda b, n: (b, 0, 0)),
            pl.BlockSpec((C2, C2), lambda b, n: (0, 0)),
            pl.BlockSpec((C1, C2), lambda b, n: (0, 0)),
            pl.BlockSpec((C2, C2), lambda b, n: (0, 0)),
            pl.BlockSpec((1, C2), lambda b, n: (0, 0)),
            pl.BlockSpec((1, C2), lambda b, n: (0, 0)),
        ],
        out_specs=pl.BlockSpec((1, BN, C2), lambda b, n: (b, n, 0)),
        out_shape=jax.ShapeDtypeStruct((B, N1, C2), jnp.float32),
    )(xyz1p, xyz2t, points1, points2, w1a, w1b, W2,
      b1.reshape(1, C2), b2.reshape(1, C2))
    return out
